# baseline (device time: 710270 ns/iter reference)
import jax
import jax.numpy as jnp
from jax import lax
from jax.experimental import pallas as pl
from jax.experimental.pallas import tpu as pltpu

N_DEV = 8
N_HOP = N_DEV - 1


def _gelu(y):
    c = 0.7978845608028654
    return 0.5 * y * (1.0 + jnp.tanh(c * (y + 0.044715 * y * y * y)))


def kernel(x, w_mat):
    m_per, k = x.shape
    _, n_per = w_mat.shape
    m_half = m_per // 2

    w_mat = w_mat.astype(jnp.bfloat16)
    conv_rows = 32
    n_conv = m_per // conv_rows

    def body(x_ref, w_ref, out_ref, cw_ref, ccw_ref, stage_ref, xb_ref,
             conv_ref, cw_send, cw_recv, ccw_send, ccw_recv,
             credit_cw, credit_ccw, exit_sem, out_sems, conv_sems):
        my = lax.axis_index("i")
        left = lax.rem(my + N_DEV - 1, N_DEV)
        right = lax.rem(my + 1, N_DEV)

        barrier_sem = pltpu.get_barrier_semaphore()
        for nbr in (left, right):
            pl.semaphore_signal(
                barrier_sem, inc=1,
                device_id=(nbr,), device_id_type=pl.DeviceIdType.MESH,
            )
        pl.semaphore_wait(barrier_sem, 2)

        def make_send(direction, src, slot):
            comm, sends, recvs, dst_dev = (
                (cw_ref, cw_send, cw_recv, right) if direction == 0
                else (ccw_ref, ccw_send, ccw_recv, left)
            )
            return pltpu.make_async_remote_copy(
                src_ref=src,
                dst_ref=comm.at[slot],
                send_sem=sends.at[slot],
                recv_sem=recvs.at[slot],
                device_id=(dst_dev,),
                device_id_type=pl.DeviceIdType.MESH,
            )

        sends_cw = []
        sends_ccw = []
        conv_copies = [None, None]

        def start_conv(c):
            cp = pltpu.make_async_copy(
                x_ref.at[pl.ds(c * conv_rows, conv_rows)],
                conv_ref.at[c % 2],
                conv_sems.at[c % 2],
            )
            cp.start()
            conv_copies[c % 2] = cp

        start_conv(0)
        start_conv(1)
        for c in range(n_conv):
            conv_copies[c % 2].wait()
            xb_ref[pl.ds(c * conv_rows, conv_rows)] = (
                conv_ref[c % 2].astype(jnp.bfloat16)
            )
            if c + 2 < n_conv:
                start_conv(c + 2)
            if c == n_conv // 2 - 1:
                sends_cw.append(make_send(0, xb_ref.at[pl.ds(0, m_half)], 0))
                sends_cw[0].start()
        sends_ccw.append(make_send(1, xb_ref.at[pl.ds(m_half, m_half)], 0))
        sends_ccw[0].start()

        out_copies = [None]

        def emit(origin, half, acc):
            if out_copies[0] is not None:
                out_copies[0].wait()
            stage_ref[...] = _gelu(acc)
            row = origin * m_per + half * m_half
            cp = pltpu.make_async_copy(
                stage_ref,
                out_ref.at[pl.ds(row, m_half)],
                out_sems,
            )
            cp.start()
            out_copies[0] = cp

        emit(my, 0, jnp.dot(xb_ref[pl.ds(0, m_half)], w_ref[...],
                            preferred_element_type=jnp.float32))
        emit(my, 1, jnp.dot(xb_ref[pl.ds(m_half, m_half)], w_ref[...],
                            preferred_element_type=jnp.float32))

        for h in range(N_HOP):
            slot = h % 2
            sends_cw[h].wait_recv()
            sends_ccw[h].wait_recv()

            if h < N_HOP - 1:
                nslot = (h + 1) % 2
                if h >= 1:
                    pl.semaphore_wait(credit_cw, 1)
                    pl.semaphore_wait(credit_ccw, 1)
                sends_cw.append(make_send(0, cw_ref.at[slot], nslot))
                sends_ccw.append(make_send(1, ccw_ref.at[slot], nslot))
                sends_cw[h + 1].start()
                sends_ccw[h + 1].start()

            o_cw = lax.rem(my + N_DEV - 1 - h, N_DEV)
            o_ccw = lax.rem(my + 1 + h, N_DEV)
            emit(o_cw, 0, jnp.dot(cw_ref[slot], w_ref[...],
                                  preferred_element_type=jnp.float32))
            emit(o_ccw, 1, jnp.dot(ccw_ref[slot], w_ref[...],
                                   preferred_element_type=jnp.float32))

            if h < N_HOP - 1:
                if h == 0:
                    sends_cw[0].wait_send()
                    sends_ccw[0].wait_send()
                sends_cw[h + 1].wait_send()
                sends_ccw[h + 1].wait_send()
                if h <= N_HOP - 3:
                    pl.semaphore_signal(
                        credit_cw, inc=1,
                        device_id=(left,), device_id_type=pl.DeviceIdType.MESH,
                    )
                    pl.semaphore_signal(
                        credit_ccw, inc=1,
                        device_id=(right,), device_id_type=pl.DeviceIdType.MESH,
                    )

        out_copies[0].wait()

        for nbr in (left, right):
            pl.semaphore_signal(
                exit_sem, inc=1,
                device_id=(nbr,), device_id_type=pl.DeviceIdType.MESH,
            )
        pl.semaphore_wait(exit_sem, 2)

    return pl.pallas_call(
        body,
        out_shape=jax.ShapeDtypeStruct((N_DEV * m_per, n_per), jnp.float32),
        in_specs=[
            pl.BlockSpec(memory_space=pltpu.MemorySpace.HBM),
            pl.BlockSpec(memory_space=pltpu.VMEM),
        ],
        out_specs=pl.BlockSpec(memory_space=pltpu.MemorySpace.HBM),
        scratch_shapes=[
            pltpu.VMEM((2, m_half, k), jnp.bfloat16),
            pltpu.VMEM((2, m_half, k), jnp.bfloat16),
            pltpu.VMEM((m_half, n_per), jnp.float32),
            pltpu.VMEM((m_per, k), jnp.bfloat16),
            pltpu.VMEM((2, conv_rows, k), jnp.float32),
            pltpu.SemaphoreType.DMA((2,)),
            pltpu.SemaphoreType.DMA((2,)),
            pltpu.SemaphoreType.DMA((2,)),
            pltpu.SemaphoreType.DMA((2,)),
            pltpu.SemaphoreType.REGULAR,
            pltpu.SemaphoreType.REGULAR,
            pltpu.SemaphoreType.REGULAR,
            pltpu.SemaphoreType.DMA,
            pltpu.SemaphoreType.DMA((2,)),
        ],
        compiler_params=pltpu.CompilerParams(
            collective_id=0, vmem_limit_bytes=63 * 1024 * 1024
        ),
    )(x, w_mat)


# device time: 700479 ns/iter; 1.0140x vs baseline; 1.0140x over previous
import jax
import jax.numpy as jnp
from jax import lax
from jax.experimental import pallas as pl
from jax.experimental.pallas import tpu as pltpu

N_DEV = 8
N_HOP = N_DEV - 1
N_SUB = 2


def _gelu(y):
    c = 0.7978845608028654
    return 0.5 * y * (1.0 + jnp.tanh(c * (y + 0.044715 * y * y * y)))


def kernel(x, w_mat):
    m_per, k = x.shape
    _, n_per = w_mat.shape
    m_half = m_per // 2
    m_sub = m_half // N_SUB

    w_mat = w_mat.astype(jnp.bfloat16)
    conv_rows = 32
    n_conv = m_per // conv_rows
    conv_per_sub = m_sub // conv_rows

    def body(x_ref, w_ref, out_ref, cw_ref, ccw_ref, stage_ref, xb_ref,
             conv_ref, cw_send, cw_recv, ccw_send, ccw_recv,
             credit_cw, credit_ccw, exit_sem, out_sems, conv_sems):
        my = lax.axis_index("i")
        left = lax.rem(my + N_DEV - 1, N_DEV)
        right = lax.rem(my + 1, N_DEV)

        barrier_sem = pltpu.get_barrier_semaphore()
        for nbr in (left, right):
            pl.semaphore_signal(
                barrier_sem, inc=1,
                device_id=(nbr,), device_id_type=pl.DeviceIdType.MESH,
            )
        pl.semaphore_wait(barrier_sem, 2)

        def make_send(direction, src, slot, sub):
            comm, sends, recvs, dst_dev = (
                (cw_ref, cw_send, cw_recv, right) if direction == 0
                else (ccw_ref, ccw_send, ccw_recv, left)
            )
            return pltpu.make_async_remote_copy(
                src_ref=src,
                dst_ref=comm.at[slot, pl.ds(sub * m_sub, m_sub)],
                send_sem=sends.at[slot, sub],
                recv_sem=recvs.at[slot, sub],
                device_id=(dst_dev,),
                device_id_type=pl.DeviceIdType.MESH,
            )

        sends_cw = [[None, None]]
        sends_ccw = [[None, None]]
        conv_copies = [None, None]

        def start_conv(c):
            cp = pltpu.make_async_copy(
                x_ref.at[pl.ds(c * conv_rows, conv_rows)],
                conv_ref.at[c % 2],
                conv_sems.at[c % 2],
            )
            cp.start()
            conv_copies[c % 2] = cp

        start_conv(0)
        start_conv(1)
        for c in range(n_conv):
            conv_copies[c % 2].wait()
            xb_ref[pl.ds(c * conv_rows, conv_rows)] = (
                conv_ref[c % 2].astype(jnp.bfloat16)
            )
            if c + 2 < n_conv:
                start_conv(c + 2)
            if (c + 1) % conv_per_sub == 0:
                piece = (c + 1) // conv_per_sub - 1
                direction, sub = divmod(piece, N_SUB)
                src = xb_ref.at[pl.ds(piece * m_sub, m_sub)]
                snd = make_send(direction, src, 0, sub)
                (sends_cw if direction == 0 else sends_ccw)[0][sub] = snd
                snd.start()

        out_copies = [None]

        def emit(origin, half, acc):
            if out_copies[0] is not None:
                out_copies[0].wait()
            stage_ref[...] = _gelu(acc)
            row = origin * m_per + half * m_half
            cp = pltpu.make_async_copy(
                stage_ref,
                out_ref.at[pl.ds(row, m_half)],
                out_sems,
            )
            cp.start()
            out_copies[0] = cp

        emit(my, 0, jnp.dot(xb_ref[pl.ds(0, m_half)], w_ref[...],
                            preferred_element_type=jnp.float32))
        emit(my, 1, jnp.dot(xb_ref[pl.ds(m_half, m_half)], w_ref[...],
                            preferred_element_type=jnp.float32))

        for h in range(N_HOP):
            slot = h % 2
            nslot = (h + 1) % 2
            fwd = h < N_HOP - 1

            sends_cw[h][0].wait_recv()
            sends_ccw[h][0].wait_recv()
            if fwd:
                if h >= 1:
                    pl.semaphore_wait(credit_cw, 1)
                    pl.semaphore_wait(credit_ccw, 1)
                sends_cw.append([None, None])
                sends_ccw.append([None, None])
                for d, (snds, comm) in enumerate(
                    ((sends_cw, cw_ref), (sends_ccw, ccw_ref))
                ):
                    snds[h + 1][0] = make_send(
                        d, comm.at[slot, pl.ds(0, m_sub)], nslot, 0
                    )
                    snds[h + 1][0].start()

            sends_cw[h][1].wait_recv()
            sends_ccw[h][1].wait_recv()
            if fwd:
                for d, (snds, comm) in enumerate(
                    ((sends_cw, cw_ref), (sends_ccw, ccw_ref))
                ):
                    snds[h + 1][1] = make_send(
                        d, comm.at[slot, pl.ds(m_sub, m_sub)], nslot, 1
                    )
                    snds[h + 1][1].start()

            o_cw = lax.rem(my + N_DEV - 1 - h, N_DEV)
            o_ccw = lax.rem(my + 1 + h, N_DEV)
            emit(o_cw, 0, jnp.dot(cw_ref[slot], w_ref[...],
                                  preferred_element_type=jnp.float32))
            emit(o_ccw, 1, jnp.dot(ccw_ref[slot], w_ref[...],
                                   preferred_element_type=jnp.float32))

            if fwd:
                if h == 0:
                    for sub in range(N_SUB):
                        sends_cw[0][sub].wait_send()
                        sends_ccw[0][sub].wait_send()
                for sub in range(N_SUB):
                    sends_cw[h + 1][sub].wait_send()
                    sends_ccw[h + 1][sub].wait_send()
                if h <= N_HOP - 3:
                    pl.semaphore_signal(
                        credit_cw, inc=1,
                        device_id=(left,), device_id_type=pl.DeviceIdType.MESH,
                    )
                    pl.semaphore_signal(
                        credit_ccw, inc=1,
                        device_id=(right,), device_id_type=pl.DeviceIdType.MESH,
                    )

        out_copies[0].wait()

        for nbr in (left, right):
            pl.semaphore_signal(
                exit_sem, inc=1,
                device_id=(nbr,), device_id_type=pl.DeviceIdType.MESH,
            )
        pl.semaphore_wait(exit_sem, 2)

    return pl.pallas_call(
        body,
        out_shape=jax.ShapeDtypeStruct((N_DEV * m_per, n_per), jnp.float32),
        in_specs=[
            pl.BlockSpec(memory_space=pltpu.MemorySpace.HBM),
            pl.BlockSpec(memory_space=pltpu.VMEM),
        ],
        out_specs=pl.BlockSpec(memory_space=pltpu.MemorySpace.HBM),
        scratch_shapes=[
            pltpu.VMEM((2, m_half, k), jnp.bfloat16),
            pltpu.VMEM((2, m_half, k), jnp.bfloat16),
            pltpu.VMEM((m_half, n_per), jnp.float32),
            pltpu.VMEM((m_per, k), jnp.bfloat16),
            pltpu.VMEM((2, conv_rows, k), jnp.float32),
            pltpu.SemaphoreType.DMA((2, N_SUB)),
            pltpu.SemaphoreType.DMA((2, N_SUB)),
            pltpu.SemaphoreType.DMA((2, N_SUB)),
            pltpu.SemaphoreType.DMA((2, N_SUB)),
            pltpu.SemaphoreType.REGULAR,
            pltpu.SemaphoreType.REGULAR,
            pltpu.SemaphoreType.REGULAR,
            pltpu.SemaphoreType.DMA,
            pltpu.SemaphoreType.DMA((2,)),
        ],
        compiler_params=pltpu.CompilerParams(
            collective_id=0, vmem_limit_bytes=63 * 1024 * 1024
        ),
    )(x, w_mat)


# device time: 683570 ns/iter; 1.0391x vs baseline; 1.0247x over previous
import jax
import jax.numpy as jnp
from jax import lax
from jax.experimental import pallas as pl
from jax.experimental.pallas import tpu as pltpu

N_DEV = 8
N_HOP = N_DEV - 1
N_SUB = 2


def _gelu(y):
    c = 0.7978845608028654
    return 0.5 * y * (1.0 + jnp.tanh(c * (y + 0.044715 * y * y * y)))


def kernel(x, w_mat):
    m_per, k = x.shape
    _, n_per = w_mat.shape
    m_half = m_per // 2
    m_sub = m_half // N_SUB

    w_mat = w_mat.astype(jnp.bfloat16)
    conv_rows = 32
    conv_per_sub = m_sub // conv_rows

    def body(x_ref, w_ref, out_ref, cw_ref, ccw_ref, stage_ref, xb_ref,
             conv_ref, cw_send, cw_recv, ccw_send, ccw_recv,
             credit_cw, credit_ccw, exit_sem, out_sems, conv_sems):
        my = lax.axis_index("i")
        left = lax.rem(my + N_DEV - 1, N_DEV)
        right = lax.rem(my + 1, N_DEV)

        barrier_sem = pltpu.get_barrier_semaphore()
        for nbr in (left, right):
            pl.semaphore_signal(
                barrier_sem, inc=1,
                device_id=(nbr,), device_id_type=pl.DeviceIdType.MESH,
            )
        pl.semaphore_wait(barrier_sem, 2)

        def make_send(direction, src, slot, sub):
            comm, sends, recvs, dst_dev = (
                (cw_ref, cw_send, cw_recv, right) if direction == 0
                else (ccw_ref, ccw_send, ccw_recv, left)
            )
            return pltpu.make_async_remote_copy(
                src_ref=src,
                dst_ref=comm.at[slot, pl.ds(sub * m_sub, m_sub)],
                send_sem=sends.at[slot, sub],
                recv_sem=recvs.at[slot, sub],
                device_id=(dst_dev,),
                device_id_type=pl.DeviceIdType.MESH,
            )

        sends_cw = [[None, None]]
        sends_ccw = [[None, None]]
        conv_copies = [None, None]
        piece_order = [0, 2, 1, 3]
        chunk_rows = [
            p * m_sub + ci * conv_rows
            for p in piece_order for ci in range(conv_per_sub)
        ]

        def start_conv(i):
            cp = pltpu.make_async_copy(
                x_ref.at[pl.ds(chunk_rows[i], conv_rows)],
                conv_ref.at[i % 2],
                conv_sems.at[i % 2],
            )
            cp.start()
            conv_copies[i % 2] = cp

        start_conv(0)
        start_conv(1)
        for i, row in enumerate(chunk_rows):
            conv_copies[i % 2].wait()
            xb_ref[pl.ds(row, conv_rows)] = conv_ref[i % 2].astype(jnp.bfloat16)
            if i + 2 < len(chunk_rows):
                start_conv(i + 2)
            if (i + 1) % conv_per_sub == 0:
                piece = piece_order[(i + 1) // conv_per_sub - 1]
                direction, sub = divmod(piece, N_SUB)
                src = xb_ref.at[pl.ds(piece * m_sub, m_sub)]
                snd = make_send(direction, src, 0, sub)
                (sends_cw if direction == 0 else sends_ccw)[0][sub] = snd
                snd.start()

        out_copies = [None]

        def emit(row, rows, acc):
            if out_copies[0] is not None:
                out_copies[0].wait()
            stage_ref[pl.ds(0, rows)] = _gelu(acc)
            cp = pltpu.make_async_copy(
                stage_ref.at[pl.ds(0, rows)],
                out_ref.at[pl.ds(row, rows)],
                out_sems,
            )
            cp.start()
            out_copies[0] = cp

        emit(my * m_per, m_half,
             jnp.dot(xb_ref[pl.ds(0, m_half)], w_ref[...],
                     preferred_element_type=jnp.float32))
        emit(my * m_per + m_half, m_half,
             jnp.dot(xb_ref[pl.ds(m_half, m_half)], w_ref[...],
                     preferred_element_type=jnp.float32))

        for h in range(N_HOP):
            slot = h % 2
            nslot = (h + 1) % 2
            fwd = h < N_HOP - 1
            o_cw = lax.rem(my + N_DEV - 1 - h, N_DEV)
            o_ccw = lax.rem(my + 1 + h, N_DEV)

            for sub in range(N_SUB):
                sends_cw[h][sub].wait_recv()
                sends_ccw[h][sub].wait_recv()
                if fwd:
                    if h == 1:
                        sends_cw[0][sub].wait_send()
                        sends_ccw[0][sub].wait_send()
                    if h >= 1:
                        pl.semaphore_wait(credit_cw.at[sub], 1)
                        pl.semaphore_wait(credit_ccw.at[sub], 1)
                    if sub == 0:
                        sends_cw.append([None, None])
                        sends_ccw.append([None, None])
                    for d, (snds, comm) in enumerate(
                        ((sends_cw, cw_ref), (sends_ccw, ccw_ref))
                    ):
                        snds[h + 1][sub] = make_send(
                            d, comm.at[slot, pl.ds(sub * m_sub, m_sub)],
                            nslot, sub,
                        )
                        snds[h + 1][sub].start()
                elif sub == 0:
                    emit(o_cw * m_per, m_sub,
                         jnp.dot(cw_ref[slot, pl.ds(0, m_sub)], w_ref[...],
                                 preferred_element_type=jnp.float32))
                    emit(o_ccw * m_per + m_half, m_sub,
                         jnp.dot(ccw_ref[slot, pl.ds(0, m_sub)], w_ref[...],
                                 preferred_element_type=jnp.float32))

            if fwd:
                emit(o_cw * m_per, m_half,
                     jnp.dot(cw_ref[slot], w_ref[...],
                             preferred_element_type=jnp.float32))
                emit(o_ccw * m_per + m_half, m_half,
                     jnp.dot(ccw_ref[slot], w_ref[...],
                             preferred_element_type=jnp.float32))
                if h <= N_HOP - 3:
                    for sub in range(N_SUB):
                        sends_cw[h + 1][sub].wait_send()
                        sends_ccw[h + 1][sub].wait_send()
                        pl.semaphore_signal(
                            credit_cw.at[sub], inc=1,
                            device_id=(left,),
                            device_id_type=pl.DeviceIdType.MESH,
                        )
                        pl.semaphore_signal(
                            credit_ccw.at[sub], inc=1,
                            device_id=(right,),
                            device_id_type=pl.DeviceIdType.MESH,
                        )
            else:
                emit(o_cw * m_per + m_sub, m_sub,
                     jnp.dot(cw_ref[slot, pl.ds(m_sub, m_sub)], w_ref[...],
                             preferred_element_type=jnp.float32))
                emit(o_ccw * m_per + m_half + m_sub, m_sub,
                     jnp.dot(ccw_ref[slot, pl.ds(m_sub, m_sub)], w_ref[...],
                             preferred_element_type=jnp.float32))

        for sub in range(N_SUB):
            sends_cw[N_HOP - 1][sub].wait_send()
            sends_ccw[N_HOP - 1][sub].wait_send()
        out_copies[0].wait()

        for nbr in (left, right):
            pl.semaphore_signal(
                exit_sem, inc=1,
                device_id=(nbr,), device_id_type=pl.DeviceIdType.MESH,
            )
        pl.semaphore_wait(exit_sem, 2)

    return pl.pallas_call(
        body,
        out_shape=jax.ShapeDtypeStruct((N_DEV * m_per, n_per), jnp.float32),
        in_specs=[
            pl.BlockSpec(memory_space=pltpu.MemorySpace.HBM),
            pl.BlockSpec(memory_space=pltpu.VMEM),
        ],
        out_specs=pl.BlockSpec(memory_space=pltpu.MemorySpace.HBM),
        scratch_shapes=[
            pltpu.VMEM((2, m_half, k), jnp.bfloat16),
            pltpu.VMEM((2, m_half, k), jnp.bfloat16),
            pltpu.VMEM((m_half, n_per), jnp.float32),
            pltpu.VMEM((m_per, k), jnp.bfloat16),
            pltpu.VMEM((2, conv_rows, k), jnp.float32),
            pltpu.SemaphoreType.DMA((2, N_SUB)),
            pltpu.SemaphoreType.DMA((2, N_SUB)),
            pltpu.SemaphoreType.DMA((2, N_SUB)),
            pltpu.SemaphoreType.DMA((2, N_SUB)),
            pltpu.SemaphoreType.REGULAR((N_SUB,)),
            pltpu.SemaphoreType.REGULAR((N_SUB,)),
            pltpu.SemaphoreType.REGULAR,
            pltpu.SemaphoreType.DMA,
            pltpu.SemaphoreType.DMA((2,)),
        ],
        compiler_params=pltpu.CompilerParams(
            collective_id=0, vmem_limit_bytes=63 * 1024 * 1024
        ),
    )(x, w_mat)
